# baseline (speedup 1.0000x reference)
import jax
import jax.numpy as jnp
from jax import lax
from jax.experimental import pallas as pl
from jax.experimental.pallas import tpu as pltpu
from jax.experimental.pallas import tpu_sc as plsc

NUM_TOKENS = 8192
D_MODEL = 32
BATCH = 1024

CHUNK = 1024
G = NUM_TOKENS // CHUNK

_NC = 2
_NS = 16
_NW = _NC * _NS
_BPW = BATCH // _NW


def _hi(v):
    return v.astype(jnp.bfloat16).astype(jnp.float32)


def _argmin_body(x_ref, cb_ref, idx_ref):
    x = x_ref[...]
    cb = cb_ref[...]
    row_iota = lax.broadcasted_iota(jnp.int32, (CHUNK, BATCH), 0)
    ones_d1 = jnp.ones((D_MODEL, 1), jnp.float32)
    ones_b1 = jnp.ones((BATCH, 1), jnp.float32)

    cn = lax.dot_general(cb * cb, ones_d1, (((1,), (0,)), ((), ())),
                         precision=lax.Precision.HIGHEST,
                         preferred_element_type=jnp.float32)
    m2c = -2.0 * cb
    m2ch = _hi(m2c)
    m2cl = m2c - m2ch
    cnh = _hi(cn)
    cnl = cn - cnh
    cb_ext = jnp.concatenate([m2ch, m2ch, m2cl, cnh, cnl], axis=1)

    xh = _hi(x)
    xl = x - xh
    x_ext = jnp.concatenate([xh, xl, xh, ones_b1, ones_b1], axis=1)

    s_all = lax.dot_general(cb_ext, x_ext, (((1,), (1,)), ((), ())),
                            preferred_element_type=jnp.float32)

    best_val = jnp.full((1, BATCH), jnp.inf, jnp.float32)
    best_idx = jnp.zeros((1, BATCH), jnp.int32)
    for t in range(G):
        s = s_all[t * CHUNK:(t + 1) * CHUNK, :]
        m = jnp.min(s, axis=0, keepdims=True)
        im = jnp.min(jnp.where(s == m, row_iota, jnp.int32(NUM_TOKENS)),
                     axis=0, keepdims=True) + t * CHUNK
        take = m < best_val
        best_val = jnp.where(take, m, best_val)
        best_idx = jnp.where(take, im, best_idx)
    idx_ref[...] = best_idx


def _gather_body(table_hbm, idx_hbm, out_hbm, idx_v, rows_v, sem):
    wid = lax.axis_index("s") * _NC + lax.axis_index("c")
    base = wid * _BPW
    pltpu.sync_copy(idx_hbm.at[0, pl.ds(base, _BPW)], idx_v)
    pltpu.async_copy(table_hbm.at[idx_v], rows_v, sem).wait()
    pltpu.sync_copy(rows_v, out_hbm.at[pl.ds(base, _BPW)])


def kernel(x, codebook, embed_table):
    argmin_call = pl.pallas_call(
        _argmin_body,
        out_shape=jax.ShapeDtypeStruct((1, BATCH), jnp.int32),
    )
    gather_call = pl.kernel(
        _gather_body,
        out_type=jax.ShapeDtypeStruct((BATCH, D_MODEL), jnp.float32),
        mesh=plsc.VectorSubcoreMesh(core_axis_name="c", subcore_axis_name="s"),
        scratch_types=[
            pltpu.VMEM((_BPW,), jnp.int32),
            pltpu.VMEM((_BPW, D_MODEL), jnp.float32),
            pltpu.SemaphoreType.DMA,
        ],
        compiler_params=pltpu.CompilerParams(use_tc_tiling_on_sc=False),
    )
    idx = argmin_call(x, codebook)
    return gather_call(embed_table, idx)
